# baseline (speedup 1.0000x reference)
import functools

import numpy as np
import jax
import jax.numpy as jnp
from jax import lax
from jax.experimental import pallas as pl
from jax.experimental.pallas import tpu as pltpu
from jax.experimental.pallas import tpu_sc as plsc

_B, _L, _D = 4096, 200, 64
_NW = 32
_K = 128
_ROWS = _B * _L
_ROWS_PER_W = _ROWS // _NW
_CHUNKS = _ROWS_PER_W // _K


def _pos_enc_np():
    pe = np.array(
        [[pos / np.power(10000, 2 * i / _D) for i in range(_D)] for pos in range(_L)],
        dtype=np.float32,
    )
    pe[:, 0::2] = np.sin(pe[:, 0::2])
    pe[:, 1::2] = np.cos(pe[:, 1::2])
    return pe


_MESH = plsc.VectorSubcoreMesh(core_axis_name="c", subcore_axis_name="s")


@functools.partial(
    pl.kernel,
    mesh=_MESH,
    compiler_params=pltpu.CompilerParams(use_tc_tiling_on_sc=False),
    out_type=jax.ShapeDtypeStruct((_ROWS, _D), jnp.float32),
    scratch_types=[
        pltpu.VMEM((_CHUNKS, _K), jnp.int32),
        pltpu.VMEM((2, _K, _D), jnp.float32),
        pltpu.VMEM((2 * _L, _D), jnp.float32),
        pltpu.SemaphoreType.DMA,
        pltpu.SemaphoreType.DMA,
    ],
)
def _embed_sc(ids_hbm, pe_hbm, table_hbm, out_hbm, idx_v, rows_v, pe_v, sem_g, sem_o):
    wid = lax.axis_index("s") * 2 + lax.axis_index("c")
    chunk0 = wid * _CHUNKS
    pltpu.sync_copy(pe_hbm, pe_v)
    pltpu.sync_copy(ids_hbm.at[pl.ds(chunk0, _CHUNKS)], idx_v)
    pltpu.async_copy(table_hbm.at[idx_v.at[0]], rows_v.at[0], sem_g)

    def chunk_body(g, carry):
        cur = g % 2
        pltpu.make_async_copy(table_hbm.at[idx_v.at[g]], rows_v.at[cur], sem_g).wait()

        @pl.when(g >= 1)
        def _wait_prev_out():
            pltpu.make_async_copy(
                rows_v.at[1 - cur],
                out_hbm.at[pl.ds((chunk0 + g - 1) * _K, _K)],
                sem_o,
            ).wait()

        @pl.when(g + 1 < _CHUNKS)
        def _start_next_gather():
            pltpu.async_copy(table_hbm.at[idx_v.at[g + 1]], rows_v.at[1 - cur], sem_g)

        p0 = ((chunk0 + g) * _K) % _L

        def add_body(i, c2):
            for j in range(_D // 16):
                sl = pl.ds(j * 16, 16)
                rows_v[cur, i, sl] = rows_v[cur, i, sl] + pe_v[p0 + i, sl]
            return c2

        lax.fori_loop(0, _K, add_body, 0, unroll=2)
        pltpu.async_copy(rows_v.at[cur], out_hbm.at[pl.ds((chunk0 + g) * _K, _K)], sem_o)
        return carry

    lax.fori_loop(0, _CHUNKS, chunk_body, 0)
    pltpu.make_async_copy(
        rows_v.at[1],
        out_hbm.at[pl.ds((chunk0 + _CHUNKS - 1) * _K, _K)],
        sem_o,
    ).wait()


def kernel(ids, table):
    ids_flat = ids.astype(jnp.int32).reshape(_ROWS // _K, _K)
    pe = jnp.asarray(np.concatenate([_pos_enc_np()] * 2, axis=0))
    out = _embed_sc(ids_flat, pe, table)
    return out.reshape(_B, _L, _D)


# speedup vs baseline: 1.0461x; 1.0461x over previous
import functools

import numpy as np
import jax
import jax.numpy as jnp
from jax import lax
from jax.experimental import pallas as pl
from jax.experimental.pallas import tpu as pltpu
from jax.experimental.pallas import tpu_sc as plsc

_B, _L, _D = 4096, 200, 64
_NW = 32
_NB = _B // _NW
_H = 104
_OFF = (0, 96)
_UNITS = 2 * _NB
_NBUF = 4


def _pos_enc_np():
    pe = np.array(
        [[pos / np.power(10000, 2 * i / _D) for i in range(_D)] for pos in range(_L)],
        dtype=np.float32,
    )
    pe[:, 0::2] = np.sin(pe[:, 0::2])
    pe[:, 1::2] = np.cos(pe[:, 1::2])
    return pe


_MESH = plsc.VectorSubcoreMesh(core_axis_name="c", subcore_axis_name="s")


@functools.partial(
    pl.kernel,
    mesh=_MESH,
    compiler_params=pltpu.CompilerParams(use_tc_tiling_on_sc=False),
    out_type=jax.ShapeDtypeStruct((_B, _L, _D), jnp.float32),
    scratch_types=[
        pltpu.VMEM((_NB, _H), jnp.int32),
        pltpu.VMEM((_NB, _H), jnp.int32),
        pltpu.VMEM((_NBUF, _H, _D), jnp.float32),
        pltpu.VMEM((_L, _D), jnp.float32),
        pltpu.SemaphoreType.DMA,
        pltpu.SemaphoreType.DMA,
    ],
)
def _embed_sc(ids_hbm, pe_hbm, table_hbm, out_hbm,
              idx_a, idx_b, rows_v, pe_v, sem_g, sem_o):
    wid = lax.axis_index("s") * 2 + lax.axis_index("c")
    b0 = wid * _NB
    pltpu.sync_copy(pe_hbm, pe_v)
    pltpu.sync_copy(ids_hbm.at[pl.ds(b0, _NB), pl.ds(_OFF[0], _H)], idx_a)
    pltpu.sync_copy(ids_hbm.at[pl.ds(b0, _NB), pl.ds(_OFF[1], _H)], idx_b)

    def unit_idx(b, half):
        return idx_a.at[b] if half == 0 else idx_b.at[b]

    def wait_out_one():
        pltpu.make_async_copy(
            rows_v.at[0], out_hbm.at[b0, pl.ds(0, _H)], sem_o
        ).wait()

    pltpu.async_copy(table_hbm.at[idx_a.at[0]], rows_v.at[0], sem_g)
    pltpu.async_copy(table_hbm.at[idx_b.at[0]], rows_v.at[1], sem_g)
    pltpu.async_copy(table_hbm.at[idx_a.at[1]], rows_v.at[2], sem_g)

    def quad_body(q, carry):
        for j in range(4):
            half = j % 2
            b = 2 * q + j // 2
            u_is_last = j == 3
            pltpu.make_async_copy(
                table_hbm.at[unit_idx(b, half)], rows_v.at[j], sem_g
            ).wait()
            nxt = (j + 1) % _NBUF

            @pl.when((4 * q + j) >= 1)
            def _wait_prev_out():
                wait_out_one()

            nj = (j + 3) % _NBUF
            nhalf = (j + 3) % 2
            nb = 2 * q + (j + 3) // 2

            if u_is_last:
                @pl.when(q + 1 < _NB // 2)
                def _start_next_gather_last():
                    pltpu.async_copy(
                        table_hbm.at[unit_idx(nb, nhalf)], rows_v.at[nj], sem_g
                    )
            else:
                @pl.when(4 * q + j + 3 < _UNITS)
                def _start_next_gather():
                    pltpu.async_copy(
                        table_hbm.at[unit_idx(nb, nhalf)], rows_v.at[nj], sem_g
                    )

            p0 = _OFF[half]

            def add_body(i, c2):
                for jj in range(_D // 16):
                    sl = pl.ds(jj * 16, 16)
                    rows_v[j, i, sl] = rows_v[j, i, sl] + pe_v[p0 + i, sl]
                return c2

            lax.fori_loop(0, _H, add_body, 0, unroll=4)
            pltpu.async_copy(
                rows_v.at[j], out_hbm.at[b0 + b, pl.ds(p0, _H)], sem_o
            )
        return carry

    lax.fori_loop(0, _NB // 2, quad_body, 0)
    wait_out_one()


def kernel(ids, table):
    pe = jnp.asarray(_pos_enc_np())
    return _embed_sc(ids.astype(jnp.int32), pe, table)


# speedup vs baseline: 1.2339x; 1.1796x over previous
import functools

import numpy as np
import jax
import jax.numpy as jnp
from jax import lax
from jax.experimental import pallas as pl
from jax.experimental.pallas import tpu as pltpu
from jax.experimental.pallas import tpu_sc as plsc

_B, _L, _D = 4096, 200, 64
_NW = 32
_NB = _B // _NW
_H = 104
_OFF = (0, 96)
_GROUPS = _NB // 2
_NBUF = 8


def _pos_enc_np():
    pe = np.array(
        [[pos / np.power(10000, 2 * i / _D) for i in range(_D)] for pos in range(_L)],
        dtype=np.float32,
    )
    pe[:, 0::2] = np.sin(pe[:, 0::2])
    pe[:, 1::2] = np.cos(pe[:, 1::2])
    return pe


_MESH = plsc.VectorSubcoreMesh(core_axis_name="c", subcore_axis_name="s")


@functools.partial(
    pl.kernel,
    mesh=_MESH,
    compiler_params=pltpu.CompilerParams(use_tc_tiling_on_sc=False),
    out_type=jax.ShapeDtypeStruct((_B, _L, _D), jnp.float32),
    scratch_types=[
        pltpu.VMEM((_NB, _H), jnp.int32),
        pltpu.VMEM((_NB, _H), jnp.int32),
        pltpu.VMEM((_NBUF, _H, _D), jnp.float32),
        pltpu.VMEM((_L, _D), jnp.float32),
        pltpu.SemaphoreType.DMA,
        pltpu.SemaphoreType.DMA,
    ],
)
def _embed_sc(ids_hbm, pe_hbm, table_hbm, out_hbm,
              idx_a, idx_b, rows_v, pe_v, sem_g, sem_o):
    wid = lax.axis_index("s") * 2 + lax.axis_index("c")
    b0 = wid * _NB
    pltpu.sync_copy(pe_hbm, pe_v)
    pltpu.sync_copy(ids_hbm.at[pl.ds(b0, _NB), pl.ds(_OFF[0], _H)], idx_a)
    pltpu.sync_copy(ids_hbm.at[pl.ds(b0, _NB), pl.ds(_OFF[1], _H)], idx_b)

    def unit_idx(b, half):
        return idx_a.at[b] if half == 0 else idx_b.at[b]

    def start_group_gathers(g, base):
        for k in range(4):
            pltpu.async_copy(
                table_hbm.at[unit_idx(2 * g + k // 2, k % 2)],
                rows_v.at[base + k],
                sem_g,
            )

    def wait_out_one():
        pltpu.make_async_copy(
            rows_v.at[0], out_hbm.at[b0, pl.ds(0, _H)], sem_o
        ).wait()

    start_group_gathers(0, 0)

    def pair_body(gg, carry):
        for parity in range(2):
            g = 2 * gg + parity
            base = parity * 4
            for k in range(4):
                pltpu.make_async_copy(
                    table_hbm.at[unit_idx(2 * g + k // 2, k % 2)],
                    rows_v.at[base + k],
                    sem_g,
                ).wait()

            def add_body(i, c2):
                for jj in range(_D // 16):
                    sl = pl.ds(jj * 16, 16)
                    pea = pe_v[_OFF[0] + i, sl]
                    peb = pe_v[_OFF[1] + i, sl]
                    rows_v[base + 0, i, sl] = rows_v[base + 0, i, sl] + pea
                    rows_v[base + 2, i, sl] = rows_v[base + 2, i, sl] + pea
                    rows_v[base + 1, i, sl] = rows_v[base + 1, i, sl] + peb
                    rows_v[base + 3, i, sl] = rows_v[base + 3, i, sl] + peb
                return c2

            lax.fori_loop(0, _H, add_body, 0, unroll=2)

            for k in range(4):
                pltpu.async_copy(
                    rows_v.at[base + k],
                    out_hbm.at[b0 + 2 * g + k // 2, pl.ds(_OFF[k % 2], _H)],
                    sem_o,
                )

            @pl.when(g >= 1)
            def _wait_prev_outs():
                for _ in range(4):
                    wait_out_one()

            @pl.when(g + 1 < _GROUPS)
            def _start_next():
                start_group_gathers(g + 1, 4 - base)
        return carry

    lax.fori_loop(0, _GROUPS // 2, pair_body, 0)
    for _ in range(4):
        wait_out_one()


def kernel(ids, table):
    pe = jnp.asarray(_pos_enc_np())
    return _embed_sc(ids.astype(jnp.int32), pe, table)
